# baseline (speedup 1.0000x reference)
import functools

import jax
import jax.numpy as jnp
from jax import lax
from jax.experimental import pallas as pl
from jax.experimental.pallas import tpu as pltpu
from jax.experimental.pallas import tpu_sc as plsc

VOCAB = 1000
EMBED = 128
B = 1024
T = 200
TOUT = T - 1
TPAD = 200

NC, NS = 2, 16
NW = NC * NS
CHUNK = 80

SEGS = (192, 320, 512)

BB = 16


def _make_sc_gather(bats):
    seg_rows = bats * TPAD
    cpw = seg_rows // (NW * CHUNK)
    rpw = cpw * CHUNK

    def body(idx_hbm, table_hbm, emb_hbm, idx_v, rows_v, bf_v, gsem, wsem):
        wid = lax.axis_index("s") * NC + lax.axis_index("c")
        base = wid * rpw
        pltpu.sync_copy(idx_hbm.at[wid], idx_v)

        pltpu.async_copy(table_hbm.at[idx_v.at[0]], rows_v.at[0], gsem.at[0])

        def convert(src, dst):
            def row(r, c):
                for j in range(EMBED // 32):
                    a = src[r, pl.ds(32 * j, 16)]
                    bh = src[r, pl.ds(32 * j + 16, 16)]
                    dst[r, pl.ds(32 * j, 32)] = plsc.pack(
                        a, bh, format=plsc.PackFormat.INTERLEAVED
                    )
                return c

            lax.fori_loop(0, CHUNK, row, 0)

        def do_chunk(g, bbuf, last):
            pltpu.make_async_copy(
                table_hbm.at[idx_v.at[g]], rows_v.at[bbuf], gsem.at[bbuf]
            ).wait()

            if not last:

                @pl.when(g + 1 < cpw)
                def _():
                    pltpu.async_copy(
                        table_hbm.at[idx_v.at[g + 1]],
                        rows_v.at[1 - bbuf],
                        gsem.at[1 - bbuf],
                    )

            @pl.when(g >= 2)
            def _():
                pltpu.make_async_copy(
                    bf_v.at[bbuf],
                    emb_hbm.at[pl.ds(base, CHUNK)],
                    wsem.at[bbuf],
                ).wait()

            convert(rows_v.at[bbuf], bf_v.at[bbuf])
            pltpu.async_copy(
                bf_v.at[bbuf],
                emb_hbm.at[pl.ds(base + g * CHUNK, CHUNK)],
                wsem.at[bbuf],
            )

        def outer(g0, carry):
            for bbuf in (0, 1):
                do_chunk(g0 * 2 + bbuf, bbuf, last=False)
            return carry

        lax.fori_loop(0, cpw // 2, outer, 0)
        if cpw % 2:
            do_chunk(jnp.int32(cpw - 1), (cpw - 1) % 2, last=True)

        for bbuf in (0, 1):
            pltpu.make_async_copy(
                bf_v.at[bbuf], emb_hbm.at[pl.ds(base, CHUNK)], wsem.at[bbuf]
            ).wait()

    return functools.partial(
        pl.kernel,
        out_type=jax.ShapeDtypeStruct((seg_rows, EMBED), jnp.bfloat16),
        mesh=plsc.VectorSubcoreMesh(
            core_axis_name="c", subcore_axis_name="s",
            num_cores=NC, num_subcores=NS,
        ),
        scratch_types=[
            pltpu.VMEM((cpw, CHUNK), jnp.int32),
            pltpu.VMEM((2, CHUNK, EMBED), jnp.float32),
            pltpu.VMEM((2, CHUNK, EMBED), jnp.bfloat16),
            pltpu.SemaphoreType.DMA((2,)),
            pltpu.SemaphoreType.DMA((2,)),
        ],
        compiler_params=pltpu.CompilerParams(needs_layout_passes=False),
    )(body)


def _mm_body_first(emb_ref, w_ref, b_ref, out_ref):
    acc = jnp.dot(emb_ref[...], w_ref[...], preferred_element_type=jnp.float32)
    acc = acc + b_ref[...]
    out_ref[...] = acc.reshape(BB, TPAD, VOCAB)[:, :TOUT, :]


def _mm_body_next(emb_ref, w_ref, b_ref, prev_ref, out_ref):
    del prev_ref
    _mm_body_first(emb_ref, w_ref, b_ref, out_ref)


def _tc_matmul(emb, w_bf16, b2d, bats, off, prev=None):
    grid = bats // BB
    in_specs = [
        pl.BlockSpec((BB * TPAD, EMBED), lambda i: (i, 0)),
        pl.BlockSpec((EMBED, VOCAB), lambda i: (0, 0)),
        pl.BlockSpec((1, VOCAB), lambda i: (0, 0)),
    ]
    args = (emb, w_bf16, b2d)
    body = _mm_body_first
    aliases = {}
    if prev is not None:
        in_specs.append(pl.BlockSpec(memory_space=pl.ANY))
        args = args + (prev,)
        body = _mm_body_next
        aliases = {3: 0}
    return pl.pallas_call(
        body,
        grid=(grid,),
        in_specs=in_specs,
        out_specs=pl.BlockSpec((BB, TOUT, VOCAB), lambda i: (i + off, 0, 0)),
        out_shape=jax.ShapeDtypeStruct((B, TOUT, VOCAB), jnp.float32),
        input_output_aliases=aliases,
    )(*args)


def _w_perm():
    perm = []
    for j in range(EMBED // 32):
        for i in range(16):
            perm.append(32 * j + i)
            perm.append(32 * j + 16 + i)
    return jnp.array(perm, dtype=jnp.int32)


def kernel(x, table, W, b):
    x = x.astype(jnp.int32)
    idx = jnp.concatenate(
        [x[:, :-1] * VOCAB + x[:, 1:], jnp.zeros((B, 1), jnp.int32)], axis=1
    )
    w_bf16 = W[_w_perm(), :].astype(jnp.bfloat16)
    b2d = b.reshape(1, VOCAB)
    embs = []
    b0 = 0
    for bats in SEGS:
        cpw = bats * TPAD // (NW * CHUNK)
        embs.append(
            _make_sc_gather(bats)(
                idx[b0:b0 + bats].reshape(NW, cpw, CHUNK), table
            )
        )
        b0 += bats
    logits = None
    b0 = 0
    for bats, emb in zip(SEGS, embs):
        logits = _tc_matmul(emb, w_bf16, b2d, bats, b0 // BB, prev=logits)
        b0 += bats
    return logits


# speedup vs baseline: 1.0102x; 1.0102x over previous
import functools

import jax
import jax.numpy as jnp
from jax import lax
from jax.experimental import pallas as pl
from jax.experimental.pallas import tpu as pltpu
from jax.experimental.pallas import tpu_sc as plsc

VOCAB = 1000
EMBED = 128
B = 1024
T = 200
TOUT = T - 1
TPAD = 200

NC, NS = 2, 16
NW = NC * NS
CHUNK = 128

SEGS = (512, 512)

BB = 16


def _make_sc_gather(bats):
    seg_rows = bats * TPAD
    cpw = seg_rows // (NW * CHUNK)
    rpw = cpw * CHUNK

    def body(idx_hbm, table_hbm, emb_hbm, idx_v, rows_v, bf_v, gsem, wsem):
        wid = lax.axis_index("s") * NC + lax.axis_index("c")
        base = wid * rpw
        pltpu.sync_copy(idx_hbm.at[wid], idx_v)

        pltpu.async_copy(table_hbm.at[idx_v.at[0]], rows_v.at[0], gsem.at[0])

        def convert(src, dst):
            def row(r, c):
                for j in range(EMBED // 32):
                    a = src[r, pl.ds(32 * j, 16)]
                    bh = src[r, pl.ds(32 * j + 16, 16)]
                    dst[r, pl.ds(32 * j, 32)] = plsc.pack(
                        a, bh, format=plsc.PackFormat.INTERLEAVED
                    )
                return c

            lax.fori_loop(0, CHUNK, row, 0)

        def do_chunk(g, bbuf, last):
            pltpu.make_async_copy(
                table_hbm.at[idx_v.at[g]], rows_v.at[bbuf], gsem.at[bbuf]
            ).wait()

            if not last:

                @pl.when(g + 1 < cpw)
                def _():
                    pltpu.async_copy(
                        table_hbm.at[idx_v.at[g + 1]],
                        rows_v.at[1 - bbuf],
                        gsem.at[1 - bbuf],
                    )

            @pl.when(g >= 2)
            def _():
                pltpu.make_async_copy(
                    bf_v.at[bbuf],
                    emb_hbm.at[pl.ds(base, CHUNK)],
                    wsem.at[bbuf],
                ).wait()

            convert(rows_v.at[bbuf], bf_v.at[bbuf])
            pltpu.async_copy(
                bf_v.at[bbuf],
                emb_hbm.at[pl.ds(base + g * CHUNK, CHUNK)],
                wsem.at[bbuf],
            )

        def outer(g0, carry):
            for bbuf in (0, 1):
                do_chunk(g0 * 2 + bbuf, bbuf, last=False)
            return carry

        lax.fori_loop(0, cpw // 2, outer, 0)
        if cpw % 2:
            do_chunk(jnp.int32(cpw - 1), (cpw - 1) % 2, last=True)

        for bbuf in (0, 1):
            pltpu.make_async_copy(
                bf_v.at[bbuf], emb_hbm.at[pl.ds(base, CHUNK)], wsem.at[bbuf]
            ).wait()

    return functools.partial(
        pl.kernel,
        out_type=jax.ShapeDtypeStruct((seg_rows, EMBED), jnp.bfloat16),
        mesh=plsc.VectorSubcoreMesh(
            core_axis_name="c", subcore_axis_name="s",
            num_cores=NC, num_subcores=NS,
        ),
        scratch_types=[
            pltpu.VMEM((cpw, CHUNK), jnp.int32),
            pltpu.VMEM((2, CHUNK, EMBED), jnp.float32),
            pltpu.VMEM((2, CHUNK, EMBED), jnp.bfloat16),
            pltpu.SemaphoreType.DMA((2,)),
            pltpu.SemaphoreType.DMA((2,)),
        ],
        compiler_params=pltpu.CompilerParams(needs_layout_passes=False),
    )(body)


def _mm_body_first(emb_ref, w_ref, b_ref, out_ref):
    acc = jnp.dot(emb_ref[...], w_ref[...], preferred_element_type=jnp.float32)
    acc = acc + b_ref[...]
    out_ref[...] = acc.reshape(BB, TPAD, VOCAB)[:, :TOUT, :]


def _mm_body_next(emb_ref, w_ref, b_ref, prev_ref, out_ref):
    del prev_ref
    _mm_body_first(emb_ref, w_ref, b_ref, out_ref)


def _tc_matmul(emb, w_bf16, b2d, bats, off, prev=None):
    grid = bats // BB
    in_specs = [
        pl.BlockSpec((BB * TPAD, EMBED), lambda i: (i, 0)),
        pl.BlockSpec((EMBED, VOCAB), lambda i: (0, 0)),
        pl.BlockSpec((1, VOCAB), lambda i: (0, 0)),
    ]
    args = (emb, w_bf16, b2d)
    body = _mm_body_first
    aliases = {}
    if prev is not None:
        in_specs.append(pl.BlockSpec(memory_space=pl.ANY))
        args = args + (prev,)
        body = _mm_body_next
        aliases = {3: 0}
    return pl.pallas_call(
        body,
        grid=(grid,),
        in_specs=in_specs,
        out_specs=pl.BlockSpec((BB, TOUT, VOCAB), lambda i: (i + off, 0, 0)),
        out_shape=jax.ShapeDtypeStruct((B, TOUT, VOCAB), jnp.float32),
        input_output_aliases=aliases,
    )(*args)


def _w_perm():
    perm = []
    for j in range(EMBED // 32):
        for i in range(16):
            perm.append(32 * j + i)
            perm.append(32 * j + 16 + i)
    return jnp.array(perm, dtype=jnp.int32)


def kernel(x, table, W, b):
    x = x.astype(jnp.int32)
    idx = jnp.concatenate(
        [x[:, :-1] * VOCAB + x[:, 1:], jnp.zeros((B, 1), jnp.int32)], axis=1
    )
    w_bf16 = W[_w_perm(), :].astype(jnp.bfloat16)
    b2d = b.reshape(1, VOCAB)
    embs = []
    b0 = 0
    for bats in SEGS:
        cpw = bats * TPAD // (NW * CHUNK)
        embs.append(
            _make_sc_gather(bats)(
                idx[b0:b0 + bats].reshape(NW, cpw, CHUNK), table
            )
        )
        b0 += bats
    logits = None
    b0 = 0
    for bats, emb in zip(SEGS, embs):
        logits = _tc_matmul(emb, w_bf16, b2d, bats, b0 // BB, prev=logits)
        b0 += bats
    return logits
